# baseline (speedup 1.0000x reference)
import functools
import math

import jax
import jax.numpy as jnp
from jax import lax
from jax.experimental import pallas as pl
from jax.experimental.pallas import tpu as pltpu
from jax.experimental.pallas import tpu_sc as plsc

NV = 10000
NC = 10000
E = 320000
HIDDEN = 64
HEADS = 4
CH = HIDDEN // HEADS
NUM_LAYERS = 2
LEVEL_VEC = math.ceil(HIDDEN / 6)
LEVEL_CON = math.ceil(HIDDEN / 2)


def _fourier(x, level):
    scales = 2.0 ** jnp.arange(-level / 2.0, level / 2.0, dtype=x.dtype)
    ms = jnp.concatenate([x / s for s in scales], axis=1)
    return jnp.concatenate([jnp.sin(ms), jnp.cos(ms)], axis=1)


try:
    _SC_INFO = plsc.get_sparse_core_info()
    _SC_CORES = _SC_INFO.num_cores
    _SC_SUBCORES = _SC_INFO.num_subcores
except Exception:
    _SC_CORES, _SC_SUBCORES = 2, 16
_NW = _SC_CORES * _SC_SUBCORES


def _gather2_kernel(per_w, chunk, d1, d2,
                    t1_hbm, i1_hbm, t2_hbm, i2_hbm, o1_hbm, o2_hbm,
                    i1_v, i2_v, r1_v, r2_v, sem1, sem2):
    wid = lax.axis_index("s") * _SC_CORES + lax.axis_index("c")
    base = wid * per_w

    def body(j, carry):
        off = base + j * chunk
        pltpu.sync_copy(i1_hbm.at[pl.ds(off, chunk)], i1_v)
        pltpu.sync_copy(i2_hbm.at[pl.ds(off, chunk)], i2_v)
        cp1 = pltpu.async_copy(t1_hbm.at[i1_v, :], r1_v, sem1)
        cp2 = pltpu.async_copy(t2_hbm.at[i2_v, :], r2_v, sem2)
        cp1.wait()
        cp2.wait()
        pltpu.sync_copy(r1_v, o1_hbm.at[pl.ds(off, chunk)])
        pltpu.sync_copy(r2_v, o2_hbm.at[pl.ds(off, chunk)])
        return carry

    lax.fori_loop(0, per_w // chunk, body, 0)


def _gather_rows2(t1, i1, t2, i2, chunk=1000):
    e = i1.shape[0]
    per_w = e // _NW
    d1 = t1.shape[1]
    d2 = t2.shape[1]
    mesh = plsc.VectorSubcoreMesh(core_axis_name="c", subcore_axis_name="s", num_cores=_SC_CORES, num_subcores=_SC_SUBCORES)
    f = functools.partial(
        pl.kernel,
        out_type=(jax.ShapeDtypeStruct((e, d1), jnp.float32),
                  jax.ShapeDtypeStruct((e, d2), jnp.float32)),
        mesh=mesh,
        scratch_types=[
            pltpu.VMEM((chunk,), jnp.int32),
            pltpu.VMEM((chunk,), jnp.int32),
            pltpu.VMEM((chunk, d1), jnp.float32),
            pltpu.VMEM((chunk, d2), jnp.float32),
            pltpu.SemaphoreType.DMA,
            pltpu.SemaphoreType.DMA,
        ],
    )(functools.partial(_gather2_kernel, per_w, chunk, d1, d2))
    return f(t1, i1.astype(jnp.int32), t2, i2.astype(jnp.int32))


_ACC_ROWS = 3328
_PASS_ROWS = 3200
_N_PASSES = 4


def _scatter_add_kernel(per_w, chunk, d,
                        ya_hbm, ia_hbm, yb_hbm, ib_hbm, out_hbm,
                        ia_v, ib_v, ia2_v, ib2_v, ra_v, rb_v, zbuf, acc, sem):
    c = lax.axis_index("c")
    s = lax.axis_index("s")
    wid = s * _SC_CORES + c
    base = wid * per_w
    rows_per_s = _ACC_ROWS // _SC_SUBCORES
    r0 = s * rows_per_s
    zv = jnp.zeros((16,), jnp.float32)

    def zbody(i, carry):
        for cc in range(d // 16):
            zbuf[i, pl.ds(cc * 16, 16)] = zv
        return carry

    lax.fori_loop(0, rows_per_s, zbody, 0)

    for p in range(_N_PASSES):
        lo = p * _PASS_ROWS
        pltpu.sync_copy(zbuf, acc.at[pl.ds(r0, rows_per_s), :])
        plsc.subcore_barrier()

        def body(j, carry):
            off = base + j * chunk
            pltpu.sync_copy(ia_hbm.at[pl.ds(off, chunk)], ia_v)
            pltpu.sync_copy(ib_hbm.at[pl.ds(off, chunk)], ib_v)
            pltpu.sync_copy(ya_hbm.at[pl.ds(off, chunk), :], ra_v)
            pltpu.sync_copy(yb_hbm.at[pl.ds(off, chunk), :], rb_v)
            for k in range(chunk // 16):
                va = ia_v[pl.ds(k * 16, 16)] - lo
                vb = ib_v[pl.ds(k * 16, 16)] - lo
                oka = (va >= 0) & (va < _PASS_ROWS)
                okb = (vb >= 0) & (vb < _PASS_ROWS)
                ia2_v[pl.ds(k * 16, 16)] = jnp.where(oka, va, _ACC_ROWS - 8)
                ib2_v[pl.ds(k * 16, 16)] = jnp.where(okb, vb, _ACC_ROWS - 8)
            pltpu.sync_copy(ra_v, acc.at[ia2_v, :], add=True)
            pltpu.sync_copy(rb_v, acc.at[ib2_v, :], add=True)
            return carry

        lax.fori_loop(0, per_w // chunk, body, 0)
        plsc.subcore_barrier()
        pltpu.sync_copy(acc.at[pl.ds(r0, rows_per_s), :],
                        out_hbm.at[c, p, pl.ds(r0, rows_per_s), :])
        plsc.subcore_barrier()


def _scatter_add_dual(y_a, idx_a, y_b, idx_b, chunk=80):
    e, d = y_a.shape
    per_w = e // _NW
    mesh = plsc.VectorSubcoreMesh(core_axis_name="c", subcore_axis_name="s", num_cores=_SC_CORES, num_subcores=_SC_SUBCORES)
    f = functools.partial(
        pl.kernel,
        out_type=pltpu.HBM((_SC_CORES, _N_PASSES, _ACC_ROWS, d), jnp.float32),
        mesh=mesh,
        scratch_types=[
            pltpu.VMEM((chunk,), jnp.int32),
            pltpu.VMEM((chunk,), jnp.int32),
            pltpu.VMEM((chunk,), jnp.int32),
            pltpu.VMEM((chunk,), jnp.int32),
            pltpu.VMEM((chunk, d), jnp.float32),
            pltpu.VMEM((chunk, d), jnp.float32),
            pltpu.VMEM((_ACC_ROWS // _SC_SUBCORES, d), jnp.float32),
            pltpu.VMEM_SHARED((_ACC_ROWS, d), jnp.float32),
            pltpu.SemaphoreType.DMA,
        ],
        compiler_params=pltpu.CompilerParams(needs_layout_passes=False),
    )(functools.partial(_scatter_add_kernel, per_w, chunk, d))
    part = f(y_a, idx_a.astype(jnp.int32), y_b, idx_b.astype(jnp.int32))
    return jnp.concatenate([part[:, p, :_PASS_ROWS, :] for p in range(_N_PASSES)], axis=1)


def _merge2_kernel(n_rows, c0, p_ref, b_ref, o_ref):
    o_ref[...] = (p_ref[0, :n_rows, c0:c0 + HIDDEN]
                  + p_ref[1, :n_rows, c0:c0 + HIDDEN] + b_ref[...])


def _merge2_bias(part, bias, n_rows, c0):
    return pl.pallas_call(
        functools.partial(_merge2_kernel, n_rows, c0),
        out_shape=jax.ShapeDtypeStruct((n_rows, HIDDEN), jnp.float32),
    )(part, bias.reshape(1, HIDDEN))


_NEG = -3.0e38


def _mdmerge_kernel(m_ref, d_ref, amax_ref, dg_ref):
    m = m_ref[...].reshape(_NW, -1)
    d = d_ref[...].reshape(_NW, -1)
    m_g = jnp.max(m, axis=0)
    scale = jnp.where(d > 0, jnp.exp(m - m_g[None]), 0.0)
    dg_ref[...] = jnp.sum(d * scale, axis=0)
    amax_ref[...] = jnp.where(m_g > -1.0e37, m_g, 0.0)


def _soft_kernel(per_w, chunk, n4,
                 al_hbm, dst_hbm, m_out, d_out,
                 dst_b, al_b, m_priv, d_priv):
    c = lax.axis_index("c")
    s = lax.axis_index("s")
    wid = s * _SC_CORES + c
    base = wid * per_w
    iota = lax.iota(jnp.int32, 16)
    mask4 = iota < 4
    sel4 = jnp.minimum(iota, 3)
    negv = jnp.full((16,), _NEG, jnp.float32)
    zv = jnp.zeros((16,), jnp.float32)

    def initb(i, carry):
        for k in range(16):
            m_priv[i, pl.ds(k * 16, 16)] = negv
            d_priv[i, pl.ds(k * 16, 16)] = zv
        return carry

    lax.fori_loop(0, n4 // 256, initb, 0)

    def load_chunk(g):
        off = base + g * chunk
        pltpu.sync_copy(dst_hbm.at[pl.ds(off, chunk)], dst_b)
        pltpu.sync_copy(al_hbm.at[pl.ds(off * 4, chunk * 4)], al_b)

    def edge_quad(gg, do_pass2):
        dstv = dst_b[pl.ds(gg * 16, 16)]
        avs = [al_b[pl.ds(gg * 64 + k * 16, 16)] for k in range(4)]
        for j in range(16):
            dstb = dstv.at[jnp.full((16,), j, jnp.int32)].get(
                mode="promise_in_bounds")
            idx = dstb * 4 + iota
            idr = lax.shift_right_logical(idx, 8)
            idc = jnp.bitwise_and(idx, 255)
            asel = avs[j // 4].at[(j % 4) * 4 + sel4].get(
                mode="promise_in_bounds")
            if do_pass2:
                mcur = plsc.load_gather(m_priv, [idr, idc])
                e = jnp.exp(asel - mcur)
                dcur = plsc.load_gather(d_priv, [idr, idc])
                plsc.store_scatter(d_priv, [idr, idc], dcur + e, mask=mask4)
            else:
                mcur = plsc.load_gather(m_priv, [idr, idc])
                plsc.store_scatter(m_priv, [idr, idc],
                                   jnp.maximum(mcur, asel), mask=mask4)

    def pass1(g, carry):
        load_chunk(g)

        def inner(gg, cc):
            edge_quad(gg, False)
            return cc

        lax.fori_loop(0, chunk // 16, inner, 0)
        return carry

    def pass2(g, carry):
        load_chunk(g)

        def inner(gg, cc):
            edge_quad(gg, True)
            return cc

        lax.fori_loop(0, chunk // 16, inner, 0)
        return carry

    lax.fori_loop(0, per_w // chunk, pass1, 0)
    lax.fori_loop(0, per_w // chunk, pass2, 0)
    pltpu.sync_copy(m_priv, m_out.at[c, s])
    pltpu.sync_copy(d_priv, d_out.at[c, s])


def _segment_softmax_stats(alpha, dst, n_rows, chunk=2000):
    e = alpha.shape[0]
    per_w = e // _NW
    n_pad = ((n_rows + 8 * _SC_SUBCORES - 1) // (8 * _SC_SUBCORES)) * 8 * _SC_SUBCORES
    n4 = n_pad * 4
    mesh = plsc.VectorSubcoreMesh(core_axis_name="c", subcore_axis_name="s",
                                  num_cores=_SC_CORES, num_subcores=_SC_SUBCORES)
    f = functools.partial(
        pl.kernel,
        out_type=(pltpu.HBM((_SC_CORES, _SC_SUBCORES, n4 // 256, 256), jnp.float32),
                  pltpu.HBM((_SC_CORES, _SC_SUBCORES, n4 // 256, 256), jnp.float32)),
        mesh=mesh,
        scratch_types=[
            pltpu.VMEM((chunk,), jnp.int32),
            pltpu.VMEM((chunk * 4,), jnp.float32),
            pltpu.VMEM((n4 // 256, 256), jnp.float32),
            pltpu.VMEM((n4 // 256, 256), jnp.float32),
        ],
        compiler_params=pltpu.CompilerParams(needs_layout_passes=False),
    )(functools.partial(_soft_kernel, per_w, chunk, n4))
    alpha_flat = alpha.reshape(-1)
    m_all, d_all = f(alpha_flat, dst.astype(jnp.int32))
    amax, d_g = pl.pallas_call(
        _mdmerge_kernel,
        out_shape=(jax.ShapeDtypeStruct((n4,), jnp.float32),
                   jax.ShapeDtypeStruct((n4,), jnp.float32)),
    )(m_all, d_all)
    amax = amax.reshape(n_pad, 4)
    d_g = d_g.reshape(n_pad, 4)
    return amax[:n_rows], d_g[:n_rows]


def _gat_alpha(xl_g, xr_g, edge_attr, p):
    xl_e = xl_g.reshape(-1, HEADS, CH)
    xr_e = xr_g.reshape(-1, HEADS, CH)
    ee = (edge_attr[:, None] @ p["We"]).reshape(-1, HEADS, CH)
    m = jax.nn.leaky_relu(xl_e + xr_e + ee, 0.2)
    return jnp.sum(m * p["att"][None], axis=-1)


def _gat_weight(xl_g, alpha, amax_e, den_e):
    w = jnp.exp(alpha - amax_e) / (den_e + 1e-16)
    xl_e = xl_g.reshape(-1, HEADS, CH)
    return (xl_e * w[:, :, None]).reshape(-1, HEADS * CH)


def _graph_norm(x, w, b, ms):
    mean = jnp.mean(x, axis=0, keepdims=True)
    out = x - mean * ms
    var = jnp.mean(out * out, axis=0, keepdims=True)
    std = jnp.sqrt(var + 1e-5)
    return w * (out / std) + b


def _identity_kernel(x_ref, o_ref):
    o_ref[...] = x_ref[...]


def _pl_identity(x):
    return pl.pallas_call(
        _identity_kernel,
        out_shape=jax.ShapeDtypeStruct(x.shape, x.dtype),
    )(x)


def kernel(objective_vector, variable_lower_bound, variable_upper_bound, constraint_lower_bound, edge_values, params, edge_index, vars_ptr, cons_ptr):
    lb = variable_lower_bound
    ub = variable_upper_bound
    lb = jnp.where(jnp.isposinf(lb), 100.0, lb)
    ub = jnp.where(jnp.isposinf(ub), 100.0, ub)
    lb = jnp.where(jnp.isneginf(lb), -100.0, lb)
    lb = jnp.where(jnp.isneginf(ub), -100.0, lb)
    x_vars = jnp.stack([objective_vector, lb, ub], axis=1)
    x_vars = _fourier(x_vars, LEVEL_VEC)
    x_cons = _fourier(constraint_lower_bound[:, None], LEVEL_CON)
    src_c = edge_index[0]
    dst_v = edge_index[1]
    zeros64 = jnp.zeros((NV, 64), jnp.float32)
    for layer in params["layers"]:
        lo, lc, lv = layer["obj"], layer["c2v"], layer["v2c"]
        p_src = jnp.concatenate([
            x_vars @ lo["Wl"] + lo["bl"],
            x_cons @ lc["Wl"] + lc["bl"],
            x_cons @ lv["Wr"] + lv["br"],
            zeros64,
        ], axis=1)
        p_dst = jnp.concatenate([
            x_vars @ lo["Wr"] + lo["br"],
            x_vars @ lc["Wr"] + lc["br"],
            x_vars @ lv["Wl"] + lv["bl"],
            zeros64,
        ], axis=1)
        g_src, g_dst = _gather_rows2(p_src, src_c, p_dst, dst_v, chunk=200)
        a_obj = _gat_alpha(g_src[:, 0:64], g_dst[:, 0:64], edge_values, lo)
        a_c2v = _gat_alpha(g_src[:, 64:128], g_dst[:, 64:128], edge_values, lc)
        a_v2c = _gat_alpha(g_dst[:, 128:192], g_src[:, 128:192], edge_values, lv)
        amax_o, den_o = _segment_softmax_stats(a_obj, dst_v, NV)
        amax_c, den_c = _segment_softmax_stats(a_c2v, dst_v, NV)
        amax_v, den_v = _segment_softmax_stats(a_v2c, src_c, NC)
        t_dst = jnp.concatenate([amax_o, den_o, amax_c, den_c,
                                 jnp.zeros((NV, 112), jnp.float32)], axis=1)
        t_src = jnp.concatenate([amax_v, den_v,
                                 jnp.zeros((NC, 120), jnp.float32)], axis=1)
        gmd_dst, gmd_src = _gather_rows2(t_dst, dst_v, t_src, src_c, chunk=400)
        y_obj = _gat_weight(g_src[:, 0:64], a_obj, gmd_dst[:, 0:4], gmd_dst[:, 4:8])
        y_c2v = _gat_weight(g_src[:, 64:128], a_c2v, gmd_dst[:, 8:12], gmd_dst[:, 12:16])
        y_v2c = _gat_weight(g_dst[:, 128:192], a_v2c, gmd_src[:, 0:4], gmd_src[:, 4:8])
        zeros_e64 = jnp.zeros((E, HIDDEN), jnp.float32)
        y_a = jnp.concatenate([y_obj + y_c2v, zeros_e64], axis=1)
        y_b = jnp.concatenate([zeros_e64, y_v2c], axis=1)
        part = _scatter_add_dual(y_a, dst_v, y_b, src_c)
        x_vars = _merge2_bias(part, lo["bias"] + lc["bias"], NV, 0)
        x_cons = _merge2_bias(part, lv["bias"], NC, HIDDEN)
        x_vars = _graph_norm(x_vars, params["gn_weight"], params["gn_bias"], params["gn_mean_scale"])
        x_cons = _graph_norm(x_cons, params["gn_weight"], params["gn_bias"], params["gn_mean_scale"])
        x_vars = jax.nn.leaky_relu(x_vars, 0.01)
        x_cons = jax.nn.leaky_relu(x_cons, 0.01)
    return (_pl_identity(x_vars), _pl_identity(x_cons))


# speedup vs baseline: 1.1825x; 1.1825x over previous
import functools
import math

import jax
import jax.numpy as jnp
from jax import lax
from jax.experimental import pallas as pl
from jax.experimental.pallas import tpu as pltpu
from jax.experimental.pallas import tpu_sc as plsc

NV = 10000
NC = 10000
E = 320000
HIDDEN = 64
HEADS = 4
CH = HIDDEN // HEADS
NUM_LAYERS = 2
LEVEL_VEC = math.ceil(HIDDEN / 6)
LEVEL_CON = math.ceil(HIDDEN / 2)


def _fourier(x, level):
    scales = 2.0 ** jnp.arange(-level / 2.0, level / 2.0, dtype=x.dtype)
    ms = jnp.concatenate([x / s for s in scales], axis=1)
    return jnp.concatenate([jnp.sin(ms), jnp.cos(ms)], axis=1)


try:
    _SC_INFO = plsc.get_sparse_core_info()
    _SC_CORES = _SC_INFO.num_cores
    _SC_SUBCORES = _SC_INFO.num_subcores
except Exception:
    _SC_CORES, _SC_SUBCORES = 2, 16
_NW = _SC_CORES * _SC_SUBCORES


def _gather2_kernel(per_w, chunk, d1, d2,
                    t1_hbm, i1_hbm, t2_hbm, i2_hbm, o1_hbm, o2_hbm,
                    i1_v, i2_v, r1_v, r2_v, sem1, sem2):
    wid = lax.axis_index("s") * _SC_CORES + lax.axis_index("c")
    base = wid * per_w

    def body(j, carry):
        off = base + j * chunk
        pltpu.sync_copy(i1_hbm.at[pl.ds(off, chunk)], i1_v)
        pltpu.sync_copy(i2_hbm.at[pl.ds(off, chunk)], i2_v)
        cp1 = pltpu.async_copy(t1_hbm.at[i1_v, :], r1_v, sem1)
        cp2 = pltpu.async_copy(t2_hbm.at[i2_v, :], r2_v, sem2)
        cp1.wait()
        cp2.wait()
        pltpu.sync_copy(r1_v, o1_hbm.at[pl.ds(off, chunk)])
        pltpu.sync_copy(r2_v, o2_hbm.at[pl.ds(off, chunk)])
        return carry

    lax.fori_loop(0, per_w // chunk, body, 0)


def _gather_rows2(t1, i1, t2, i2, chunk=1000):
    e = i1.shape[0]
    per_w = e // _NW
    d1 = t1.shape[1]
    d2 = t2.shape[1]
    mesh = plsc.VectorSubcoreMesh(core_axis_name="c", subcore_axis_name="s", num_cores=_SC_CORES, num_subcores=_SC_SUBCORES)
    f = functools.partial(
        pl.kernel,
        out_type=(jax.ShapeDtypeStruct((e, d1), jnp.float32),
                  jax.ShapeDtypeStruct((e, d2), jnp.float32)),
        mesh=mesh,
        scratch_types=[
            pltpu.VMEM((chunk,), jnp.int32),
            pltpu.VMEM((chunk,), jnp.int32),
            pltpu.VMEM((chunk, d1), jnp.float32),
            pltpu.VMEM((chunk, d2), jnp.float32),
            pltpu.SemaphoreType.DMA,
            pltpu.SemaphoreType.DMA,
        ],
    )(functools.partial(_gather2_kernel, per_w, chunk, d1, d2))
    return f(t1, i1.astype(jnp.int32), t2, i2.astype(jnp.int32))


_ACC_ROWS = 3328
_PASS_ROWS = 3200
_N_PASSES = 4


def _scatter_add_kernel(per_w, chunk, d,
                        ya_hbm, ia_hbm, yb_hbm, ib_hbm, out_hbm,
                        ia_v, ib_v, ia2_v, ib2_v, ra_v, rb_v, zbuf, acc, sem):
    c = lax.axis_index("c")
    s = lax.axis_index("s")
    wid = s * _SC_CORES + c
    base = wid * per_w
    rows_per_s = _ACC_ROWS // _SC_SUBCORES
    r0 = s * rows_per_s
    zv = jnp.zeros((16,), jnp.float32)

    def zbody(i, carry):
        for cc in range(d // 16):
            zbuf[i, pl.ds(cc * 16, 16)] = zv
        return carry

    lax.fori_loop(0, rows_per_s, zbody, 0)

    for p in range(_N_PASSES):
        lo = p * _PASS_ROWS
        pltpu.sync_copy(zbuf, acc.at[pl.ds(r0, rows_per_s), :])
        plsc.subcore_barrier()

        def body(j, carry):
            off = base + j * chunk
            pltpu.sync_copy(ia_hbm.at[pl.ds(off, chunk)], ia_v)
            pltpu.sync_copy(ib_hbm.at[pl.ds(off, chunk)], ib_v)
            pltpu.sync_copy(ya_hbm.at[pl.ds(off, chunk), :], ra_v)
            pltpu.sync_copy(yb_hbm.at[pl.ds(off, chunk), :], rb_v)
            for k in range(chunk // 16):
                va = ia_v[pl.ds(k * 16, 16)] - lo
                vb = ib_v[pl.ds(k * 16, 16)] - lo
                oka = (va >= 0) & (va < _PASS_ROWS)
                okb = (vb >= 0) & (vb < _PASS_ROWS)
                ia2_v[pl.ds(k * 16, 16)] = jnp.where(oka, va, _ACC_ROWS - 8)
                ib2_v[pl.ds(k * 16, 16)] = jnp.where(okb, vb, _ACC_ROWS - 8)
            pltpu.sync_copy(ra_v, acc.at[ia2_v, :], add=True)
            pltpu.sync_copy(rb_v, acc.at[ib2_v, :], add=True)
            return carry

        lax.fori_loop(0, per_w // chunk, body, 0)
        plsc.subcore_barrier()
        pltpu.sync_copy(acc.at[pl.ds(r0, rows_per_s), :],
                        out_hbm.at[c, p, pl.ds(r0, rows_per_s), :])
        plsc.subcore_barrier()


def _scatter_add_dual(y_a, idx_a, y_b, idx_b, chunk=400):
    e, d = y_a.shape
    per_w = e // _NW
    mesh = plsc.VectorSubcoreMesh(core_axis_name="c", subcore_axis_name="s", num_cores=_SC_CORES, num_subcores=_SC_SUBCORES)
    f = functools.partial(
        pl.kernel,
        out_type=pltpu.HBM((_SC_CORES, _N_PASSES, _ACC_ROWS, d), jnp.float32),
        mesh=mesh,
        scratch_types=[
            pltpu.VMEM((chunk,), jnp.int32),
            pltpu.VMEM((chunk,), jnp.int32),
            pltpu.VMEM((chunk,), jnp.int32),
            pltpu.VMEM((chunk,), jnp.int32),
            pltpu.VMEM((chunk, d), jnp.float32),
            pltpu.VMEM((chunk, d), jnp.float32),
            pltpu.VMEM((_ACC_ROWS // _SC_SUBCORES, d), jnp.float32),
            pltpu.VMEM_SHARED((_ACC_ROWS, d), jnp.float32),
            pltpu.SemaphoreType.DMA,
        ],
        compiler_params=pltpu.CompilerParams(needs_layout_passes=False),
    )(functools.partial(_scatter_add_kernel, per_w, chunk, d))
    part = f(y_a, idx_a.astype(jnp.int32), y_b, idx_b.astype(jnp.int32))
    return jnp.concatenate([part[:, p, :_PASS_ROWS, :] for p in range(_N_PASSES)], axis=1)


def _merge2_kernel(n_rows, c0, p_ref, b_ref, o_ref):
    o_ref[...] = (p_ref[0, :n_rows, c0:c0 + HIDDEN]
                  + p_ref[1, :n_rows, c0:c0 + HIDDEN] + b_ref[...])


def _merge2_bias(part, bias, n_rows, c0):
    return pl.pallas_call(
        functools.partial(_merge2_kernel, n_rows, c0),
        out_shape=jax.ShapeDtypeStruct((n_rows, HIDDEN), jnp.float32),
    )(part, bias.reshape(1, HIDDEN))


_NEG = -3.0e38


def _mdmerge_kernel(m_ref, d_ref, amax_ref, dg_ref):
    m = m_ref[...].reshape(_NW, -1)
    d = d_ref[...].reshape(_NW, -1)
    m_g = jnp.max(m, axis=0)
    scale = jnp.where(d > 0, jnp.exp(m - m_g[None]), 0.0)
    dg_ref[...] = jnp.sum(d * scale, axis=0)
    amax_ref[...] = jnp.where(m_g > -1.0e37, m_g, 0.0)


def _soft_kernel(per_w, chunk, n4,
                 al_hbm, dst_hbm, m_out, d_out,
                 dst_b, al_b, m_priv, d_priv):
    c = lax.axis_index("c")
    s = lax.axis_index("s")
    wid = s * _SC_CORES + c
    base = wid * per_w
    iota = lax.iota(jnp.int32, 16)
    mask4 = iota < 4
    sel4 = jnp.minimum(iota, 3)
    negv = jnp.full((16,), _NEG, jnp.float32)
    zv = jnp.zeros((16,), jnp.float32)

    def initb(i, carry):
        for k in range(16):
            m_priv[i, pl.ds(k * 16, 16)] = negv
            d_priv[i, pl.ds(k * 16, 16)] = zv
        return carry

    lax.fori_loop(0, n4 // 256, initb, 0)

    def load_chunk(g):
        off = base + g * chunk
        pltpu.sync_copy(dst_hbm.at[pl.ds(off, chunk)], dst_b)
        pltpu.sync_copy(al_hbm.at[pl.ds(off * 4, chunk * 4)], al_b)

    rot4 = jnp.bitwise_and(iota + 4, 15)
    rot8 = jnp.bitwise_and(iota + 8, 15)
    g4 = lax.shift_right_logical(iota, 2)
    h4 = jnp.bitwise_and(iota, 3)

    def edge_quad(gg, do_pass2):
        dstv = dst_b[pl.ds(gg * 16, 16)]

        for q in range(4):
            dq = dstv.at[q * 4 + g4].get(mode="promise_in_bounds")
            aq = al_b[pl.ds(gg * 64 + q * 16, 16)]
            idx = dq * 4 + h4
            idr = lax.shift_right_logical(idx, 8)
            idc = jnp.bitwise_and(idx, 255)
            c1 = dq == dq.at[rot4].get(mode="promise_in_bounds")
            c2 = dq == dq.at[rot8].get(mode="promise_in_bounds")
            conflict = jnp.any(c1 | c2)

            def rmw(mask):
                if do_pass2:
                    mcur = plsc.load_gather(m_priv, [idr, idc])
                    e = jnp.exp(aq - mcur)
                    dcur = plsc.load_gather(d_priv, [idr, idc])
                    plsc.store_scatter(d_priv, [idr, idc], dcur + e,
                                       mask=mask)
                else:
                    mcur = plsc.load_gather(m_priv, [idr, idc])
                    plsc.store_scatter(m_priv, [idr, idc],
                                       jnp.maximum(mcur, aq), mask=mask)

            def fast(carry):
                rmw(None)
                return carry

            def slow(carry):
                for k in range(4):
                    rmw(g4 == k)
                return carry

            lax.cond(conflict, slow, fast, 0)

    def pass1(g, carry):
        load_chunk(g)

        def inner(gg, cc):
            edge_quad(gg, False)
            return cc

        lax.fori_loop(0, chunk // 16, inner, 0)
        return carry

    def pass2(g, carry):
        load_chunk(g)

        def inner(gg, cc):
            edge_quad(gg, True)
            return cc

        lax.fori_loop(0, chunk // 16, inner, 0)
        return carry

    lax.fori_loop(0, per_w // chunk, pass1, 0)
    lax.fori_loop(0, per_w // chunk, pass2, 0)
    pltpu.sync_copy(m_priv, m_out.at[c, s])
    pltpu.sync_copy(d_priv, d_out.at[c, s])


def _segment_softmax_stats(alpha, dst, n_rows, chunk=2000):
    e = alpha.shape[0]
    per_w = e // _NW
    n_pad = ((n_rows + 8 * _SC_SUBCORES - 1) // (8 * _SC_SUBCORES)) * 8 * _SC_SUBCORES
    n4 = n_pad * 4
    mesh = plsc.VectorSubcoreMesh(core_axis_name="c", subcore_axis_name="s",
                                  num_cores=_SC_CORES, num_subcores=_SC_SUBCORES)
    f = functools.partial(
        pl.kernel,
        out_type=(pltpu.HBM((_SC_CORES, _SC_SUBCORES, n4 // 256, 256), jnp.float32),
                  pltpu.HBM((_SC_CORES, _SC_SUBCORES, n4 // 256, 256), jnp.float32)),
        mesh=mesh,
        scratch_types=[
            pltpu.VMEM((chunk,), jnp.int32),
            pltpu.VMEM((chunk * 4,), jnp.float32),
            pltpu.VMEM((n4 // 256, 256), jnp.float32),
            pltpu.VMEM((n4 // 256, 256), jnp.float32),
        ],
        compiler_params=pltpu.CompilerParams(needs_layout_passes=False),
    )(functools.partial(_soft_kernel, per_w, chunk, n4))
    alpha_flat = alpha.reshape(-1)
    m_all, d_all = f(alpha_flat, dst.astype(jnp.int32))
    amax, d_g = pl.pallas_call(
        _mdmerge_kernel,
        out_shape=(jax.ShapeDtypeStruct((n4,), jnp.float32),
                   jax.ShapeDtypeStruct((n4,), jnp.float32)),
    )(m_all, d_all)
    amax = amax.reshape(n_pad, 4)
    d_g = d_g.reshape(n_pad, 4)
    return amax[:n_rows], d_g[:n_rows]


def _gat_alpha(xl_g, xr_g, edge_attr, p):
    xl_e = xl_g.reshape(-1, HEADS, CH)
    xr_e = xr_g.reshape(-1, HEADS, CH)
    ee = (edge_attr[:, None] @ p["We"]).reshape(-1, HEADS, CH)
    m = jax.nn.leaky_relu(xl_e + xr_e + ee, 0.2)
    return jnp.sum(m * p["att"][None], axis=-1)


def _gat_weight(xl_g, alpha, amax_e, den_e):
    w = jnp.exp(alpha - amax_e) / (den_e + 1e-16)
    xl_e = xl_g.reshape(-1, HEADS, CH)
    return (xl_e * w[:, :, None]).reshape(-1, HEADS * CH)


def _graph_norm(x, w, b, ms):
    mean = jnp.mean(x, axis=0, keepdims=True)
    out = x - mean * ms
    var = jnp.mean(out * out, axis=0, keepdims=True)
    std = jnp.sqrt(var + 1e-5)
    return w * (out / std) + b


def _identity_kernel(x_ref, o_ref):
    o_ref[...] = x_ref[...]


def _pl_identity(x):
    return pl.pallas_call(
        _identity_kernel,
        out_shape=jax.ShapeDtypeStruct(x.shape, x.dtype),
    )(x)


def kernel(objective_vector, variable_lower_bound, variable_upper_bound, constraint_lower_bound, edge_values, params, edge_index, vars_ptr, cons_ptr):
    lb = variable_lower_bound
    ub = variable_upper_bound
    lb = jnp.where(jnp.isposinf(lb), 100.0, lb)
    ub = jnp.where(jnp.isposinf(ub), 100.0, ub)
    lb = jnp.where(jnp.isneginf(lb), -100.0, lb)
    lb = jnp.where(jnp.isneginf(ub), -100.0, lb)
    x_vars = jnp.stack([objective_vector, lb, ub], axis=1)
    x_vars = _fourier(x_vars, LEVEL_VEC)
    x_cons = _fourier(constraint_lower_bound[:, None], LEVEL_CON)
    src_c = edge_index[0]
    dst_v = edge_index[1]
    zeros64 = jnp.zeros((NV, 64), jnp.float32)
    for layer in params["layers"]:
        lo, lc, lv = layer["obj"], layer["c2v"], layer["v2c"]
        p_src = jnp.concatenate([
            x_vars @ lo["Wl"] + lo["bl"],
            x_cons @ lc["Wl"] + lc["bl"],
            x_cons @ lv["Wr"] + lv["br"],
            zeros64,
        ], axis=1)
        p_dst = jnp.concatenate([
            x_vars @ lo["Wr"] + lo["br"],
            x_vars @ lc["Wr"] + lc["br"],
            x_vars @ lv["Wl"] + lv["bl"],
            zeros64,
        ], axis=1)
        g_src, g_dst = _gather_rows2(p_src, src_c, p_dst, dst_v, chunk=200)
        a_obj = _gat_alpha(g_src[:, 0:64], g_dst[:, 0:64], edge_values, lo)
        a_c2v = _gat_alpha(g_src[:, 64:128], g_dst[:, 64:128], edge_values, lc)
        a_v2c = _gat_alpha(g_dst[:, 128:192], g_src[:, 128:192], edge_values, lv)
        amax_o, den_o = _segment_softmax_stats(a_obj, dst_v, NV)
        amax_c, den_c = _segment_softmax_stats(a_c2v, dst_v, NV)
        amax_v, den_v = _segment_softmax_stats(a_v2c, src_c, NC)
        t_dst = jnp.concatenate([amax_o, den_o, amax_c, den_c,
                                 jnp.zeros((NV, 112), jnp.float32)], axis=1)
        t_src = jnp.concatenate([amax_v, den_v,
                                 jnp.zeros((NC, 120), jnp.float32)], axis=1)
        gmd_dst, gmd_src = _gather_rows2(t_dst, dst_v, t_src, src_c, chunk=400)
        y_obj = _gat_weight(g_src[:, 0:64], a_obj, gmd_dst[:, 0:4], gmd_dst[:, 4:8])
        y_c2v = _gat_weight(g_src[:, 64:128], a_c2v, gmd_dst[:, 8:12], gmd_dst[:, 12:16])
        y_v2c = _gat_weight(g_dst[:, 128:192], a_v2c, gmd_src[:, 0:4], gmd_src[:, 4:8])
        x_vars = jax.ops.segment_sum(y_obj + y_c2v, dst_v, num_segments=NV) + lo["bias"] + lc["bias"]
        x_cons = jax.ops.segment_sum(y_v2c, src_c, num_segments=NC) + lv["bias"]
        x_vars = _graph_norm(x_vars, params["gn_weight"], params["gn_bias"], params["gn_mean_scale"])
        x_cons = _graph_norm(x_cons, params["gn_weight"], params["gn_bias"], params["gn_mean_scale"])
        x_vars = jax.nn.leaky_relu(x_vars, 0.01)
        x_cons = jax.nn.leaky_relu(x_cons, 0.01)
    return (_pl_identity(x_vars), _pl_identity(x_cons))


# speedup vs baseline: 1.2121x; 1.0251x over previous
import functools
import math

import jax
import jax.numpy as jnp
from jax import lax
from jax.experimental import pallas as pl
from jax.experimental.pallas import tpu as pltpu
from jax.experimental.pallas import tpu_sc as plsc

NV = 10000
NC = 10000
E = 320000
HIDDEN = 64
HEADS = 4
CH = HIDDEN // HEADS
NUM_LAYERS = 2
LEVEL_VEC = math.ceil(HIDDEN / 6)
LEVEL_CON = math.ceil(HIDDEN / 2)


def _fourier(x, level):
    scales = 2.0 ** jnp.arange(-level / 2.0, level / 2.0, dtype=x.dtype)
    ms = jnp.concatenate([x / s for s in scales], axis=1)
    return jnp.concatenate([jnp.sin(ms), jnp.cos(ms)], axis=1)


try:
    _SC_INFO = plsc.get_sparse_core_info()
    _SC_CORES = _SC_INFO.num_cores
    _SC_SUBCORES = _SC_INFO.num_subcores
except Exception:
    _SC_CORES, _SC_SUBCORES = 2, 16
_NW = _SC_CORES * _SC_SUBCORES


def _gather2_kernel(per_w, chunk, d1, d2,
                    t1_hbm, i1_hbm, t2_hbm, i2_hbm, o1_hbm, o2_hbm,
                    i1_v, i2_v, r1_v, r2_v, sem1, sem2):
    wid = lax.axis_index("s") * _SC_CORES + lax.axis_index("c")
    base = wid * per_w

    def body(j, carry):
        off = base + j * chunk
        pltpu.sync_copy(i1_hbm.at[pl.ds(off, chunk)], i1_v)
        pltpu.sync_copy(i2_hbm.at[pl.ds(off, chunk)], i2_v)
        cp1 = pltpu.async_copy(t1_hbm.at[i1_v, :], r1_v, sem1)
        cp2 = pltpu.async_copy(t2_hbm.at[i2_v, :], r2_v, sem2)
        cp1.wait()
        cp2.wait()
        pltpu.sync_copy(r1_v, o1_hbm.at[pl.ds(off, chunk)])
        pltpu.sync_copy(r2_v, o2_hbm.at[pl.ds(off, chunk)])
        return carry

    lax.fori_loop(0, per_w // chunk, body, 0)


def _gather_rows2(t1, i1, t2, i2, chunk=1000):
    e = i1.shape[0]
    per_w = e // _NW
    d1 = t1.shape[1]
    d2 = t2.shape[1]
    mesh = plsc.VectorSubcoreMesh(core_axis_name="c", subcore_axis_name="s", num_cores=_SC_CORES, num_subcores=_SC_SUBCORES)
    f = functools.partial(
        pl.kernel,
        out_type=(jax.ShapeDtypeStruct((e, d1), jnp.float32),
                  jax.ShapeDtypeStruct((e, d2), jnp.float32)),
        mesh=mesh,
        scratch_types=[
            pltpu.VMEM((chunk,), jnp.int32),
            pltpu.VMEM((chunk,), jnp.int32),
            pltpu.VMEM((chunk, d1), jnp.float32),
            pltpu.VMEM((chunk, d2), jnp.float32),
            pltpu.SemaphoreType.DMA,
            pltpu.SemaphoreType.DMA,
        ],
    )(functools.partial(_gather2_kernel, per_w, chunk, d1, d2))
    return f(t1, i1.astype(jnp.int32), t2, i2.astype(jnp.int32))


_ACC_ROWS = 3328
_PASS_ROWS = 3200
_N_PASSES = 4


def _scatter_add_kernel(per_w, chunk, d,
                        ya_hbm, ia_hbm, yb_hbm, ib_hbm, out_hbm,
                        ia_v, ib_v, ia2_v, ib2_v, ra_v, rb_v, zbuf, acc, sem):
    c = lax.axis_index("c")
    s = lax.axis_index("s")
    wid = s * _SC_CORES + c
    base = wid * per_w
    rows_per_s = _ACC_ROWS // _SC_SUBCORES
    r0 = s * rows_per_s
    zv = jnp.zeros((16,), jnp.float32)

    def zbody(i, carry):
        for cc in range(d // 16):
            zbuf[i, pl.ds(cc * 16, 16)] = zv
        return carry

    lax.fori_loop(0, rows_per_s, zbody, 0)

    for p in range(_N_PASSES):
        lo = p * _PASS_ROWS
        pltpu.sync_copy(zbuf, acc.at[pl.ds(r0, rows_per_s), :])
        plsc.subcore_barrier()

        def body(j, carry):
            off = base + j * chunk
            pltpu.sync_copy(ia_hbm.at[pl.ds(off, chunk)], ia_v)
            pltpu.sync_copy(ib_hbm.at[pl.ds(off, chunk)], ib_v)
            pltpu.sync_copy(ya_hbm.at[pl.ds(off, chunk), :], ra_v)
            pltpu.sync_copy(yb_hbm.at[pl.ds(off, chunk), :], rb_v)
            for k in range(chunk // 16):
                va = ia_v[pl.ds(k * 16, 16)] - lo
                vb = ib_v[pl.ds(k * 16, 16)] - lo
                oka = (va >= 0) & (va < _PASS_ROWS)
                okb = (vb >= 0) & (vb < _PASS_ROWS)
                ia2_v[pl.ds(k * 16, 16)] = jnp.where(oka, va, _ACC_ROWS - 8)
                ib2_v[pl.ds(k * 16, 16)] = jnp.where(okb, vb, _ACC_ROWS - 8)
            pltpu.sync_copy(ra_v, acc.at[ia2_v, :], add=True)
            pltpu.sync_copy(rb_v, acc.at[ib2_v, :], add=True)
            return carry

        lax.fori_loop(0, per_w // chunk, body, 0)
        plsc.subcore_barrier()
        pltpu.sync_copy(acc.at[pl.ds(r0, rows_per_s), :],
                        out_hbm.at[c, p, pl.ds(r0, rows_per_s), :])
        plsc.subcore_barrier()


def _scatter_add_dual(y_a, idx_a, y_b, idx_b, chunk=400):
    e, d = y_a.shape
    per_w = e // _NW
    mesh = plsc.VectorSubcoreMesh(core_axis_name="c", subcore_axis_name="s", num_cores=_SC_CORES, num_subcores=_SC_SUBCORES)
    f = functools.partial(
        pl.kernel,
        out_type=pltpu.HBM((_SC_CORES, _N_PASSES, _ACC_ROWS, d), jnp.float32),
        mesh=mesh,
        scratch_types=[
            pltpu.VMEM((chunk,), jnp.int32),
            pltpu.VMEM((chunk,), jnp.int32),
            pltpu.VMEM((chunk,), jnp.int32),
            pltpu.VMEM((chunk,), jnp.int32),
            pltpu.VMEM((chunk, d), jnp.float32),
            pltpu.VMEM((chunk, d), jnp.float32),
            pltpu.VMEM((_ACC_ROWS // _SC_SUBCORES, d), jnp.float32),
            pltpu.VMEM_SHARED((_ACC_ROWS, d), jnp.float32),
            pltpu.SemaphoreType.DMA,
        ],
        compiler_params=pltpu.CompilerParams(needs_layout_passes=False),
    )(functools.partial(_scatter_add_kernel, per_w, chunk, d))
    part = f(y_a, idx_a.astype(jnp.int32), y_b, idx_b.astype(jnp.int32))
    return jnp.concatenate([part[:, p, :_PASS_ROWS, :] for p in range(_N_PASSES)], axis=1)


def _merge2_kernel(n_rows, c0, p_ref, b_ref, o_ref):
    o_ref[...] = (p_ref[0, :n_rows, c0:c0 + HIDDEN]
                  + p_ref[1, :n_rows, c0:c0 + HIDDEN] + b_ref[...])


def _merge2_bias(part, bias, n_rows, c0):
    return pl.pallas_call(
        functools.partial(_merge2_kernel, n_rows, c0),
        out_shape=jax.ShapeDtypeStruct((n_rows, HIDDEN), jnp.float32),
    )(part, bias.reshape(1, HIDDEN))


_NEG = -3.0e38


def _mdmerge_kernel(m_ref, d_ref, amax_ref, dg_ref):
    m = m_ref[...].reshape(_NW, -1)
    d = d_ref[...].reshape(_NW, -1)
    m_g = jnp.max(m, axis=0)
    scale = jnp.where(d > 0, jnp.exp(m - m_g[None]), 0.0)
    dg_ref[...] = jnp.sum(d * scale, axis=0)
    amax_ref[...] = jnp.where(m_g > -1.0e37, m_g, 0.0)


def _soft_kernel(per_w, chunk, n4,
                 al_hbm, dst_hbm, m_out, d_out,
                 dst_b, al_b, m_priv, d_priv):
    c = lax.axis_index("c")
    s = lax.axis_index("s")
    wid = s * _SC_CORES + c
    base = wid * per_w
    iota = lax.iota(jnp.int32, 16)
    mask4 = iota < 4
    sel4 = jnp.minimum(iota, 3)
    negv = jnp.full((16,), _NEG, jnp.float32)
    zv = jnp.zeros((16,), jnp.float32)

    def initb(i, carry):
        for k in range(16):
            m_priv[i, pl.ds(k * 16, 16)] = negv
            d_priv[i, pl.ds(k * 16, 16)] = zv
        return carry

    lax.fori_loop(0, n4 // 256, initb, 0)

    def load_chunk(g):
        off = base + g * chunk
        pltpu.sync_copy(dst_hbm.at[pl.ds(off, chunk)], dst_b)
        pltpu.sync_copy(al_hbm.at[pl.ds(off * 4, chunk * 4)], al_b)

    rot4 = jnp.bitwise_and(iota + 4, 15)
    rot8 = jnp.bitwise_and(iota + 8, 15)
    g4 = lax.shift_right_logical(iota, 2)
    h4 = jnp.bitwise_and(iota, 3)

    def edge_quad(gg, do_pass2):
        dstv = dst_b[pl.ds(gg * 16, 16)]

        for q in range(4):
            dq = dstv.at[q * 4 + g4].get(mode="promise_in_bounds")
            aq = al_b[pl.ds(gg * 64 + q * 16, 16)]
            idx = dq * 4 + h4
            idr = lax.shift_right_logical(idx, 8)
            idc = jnp.bitwise_and(idx, 255)
            c1 = dq == dq.at[rot4].get(mode="promise_in_bounds")
            c2 = dq == dq.at[rot8].get(mode="promise_in_bounds")
            conflict = jnp.any(c1 | c2)

            def rmw(mask):
                if do_pass2:
                    mcur = plsc.load_gather(m_priv, [idr, idc])
                    e = jnp.exp(aq - mcur)
                    dcur = plsc.load_gather(d_priv, [idr, idc])
                    plsc.store_scatter(d_priv, [idr, idc], dcur + e,
                                       mask=mask)
                else:
                    mcur = plsc.load_gather(m_priv, [idr, idc])
                    plsc.store_scatter(m_priv, [idr, idc],
                                       jnp.maximum(mcur, aq), mask=mask)

            def fast(carry):
                rmw(None)
                return carry

            def slow(carry):
                for k in range(4):
                    rmw(g4 == k)
                return carry

            lax.cond(conflict, slow, fast, 0)

    def pass1(g, carry):
        load_chunk(g)

        def inner(gg, cc):
            edge_quad(gg, False)
            return cc

        lax.fori_loop(0, chunk // 16, inner, 0)
        return carry

    def pass2(g, carry):
        load_chunk(g)

        def inner(gg, cc):
            edge_quad(gg, True)
            return cc

        lax.fori_loop(0, chunk // 16, inner, 0)
        return carry

    lax.fori_loop(0, per_w // chunk, pass1, 0)
    lax.fori_loop(0, per_w // chunk, pass2, 0)
    pltpu.sync_copy(m_priv, m_out.at[c, s])
    pltpu.sync_copy(d_priv, d_out.at[c, s])


def _segment_softmax_stats(alpha, dst, n_rows, chunk=2000):
    e = alpha.shape[0]
    per_w = e // _NW
    n_pad = ((n_rows + 8 * _SC_SUBCORES - 1) // (8 * _SC_SUBCORES)) * 8 * _SC_SUBCORES
    n4 = n_pad * 4
    mesh = plsc.VectorSubcoreMesh(core_axis_name="c", subcore_axis_name="s",
                                  num_cores=_SC_CORES, num_subcores=_SC_SUBCORES)
    f = functools.partial(
        pl.kernel,
        out_type=(pltpu.HBM((_SC_CORES, _SC_SUBCORES, n4 // 256, 256), jnp.float32),
                  pltpu.HBM((_SC_CORES, _SC_SUBCORES, n4 // 256, 256), jnp.float32)),
        mesh=mesh,
        scratch_types=[
            pltpu.VMEM((chunk,), jnp.int32),
            pltpu.VMEM((chunk * 4,), jnp.float32),
            pltpu.VMEM((n4 // 256, 256), jnp.float32),
            pltpu.VMEM((n4 // 256, 256), jnp.float32),
        ],
        compiler_params=pltpu.CompilerParams(needs_layout_passes=False),
    )(functools.partial(_soft_kernel, per_w, chunk, n4))
    alpha_flat = alpha.reshape(-1)
    m_all, d_all = f(alpha_flat, dst.astype(jnp.int32))
    amax, d_g = pl.pallas_call(
        _mdmerge_kernel,
        out_shape=(jax.ShapeDtypeStruct((n4,), jnp.float32),
                   jax.ShapeDtypeStruct((n4,), jnp.float32)),
    )(m_all, d_all)
    amax = amax.reshape(n_pad, 4)
    d_g = d_g.reshape(n_pad, 4)
    return amax[:n_rows], d_g[:n_rows]


def _gat_alpha(xl_g, xr_g, edge_attr, p):
    xl_e = xl_g.reshape(-1, HEADS, CH)
    xr_e = xr_g.reshape(-1, HEADS, CH)
    ee = (edge_attr[:, None] @ p["We"]).reshape(-1, HEADS, CH)
    m = jax.nn.leaky_relu(xl_e + xr_e + ee, 0.2)
    return jnp.sum(m * p["att"][None], axis=-1)


def _gat_weight(xl_g, alpha, amax_e, den_e):
    w = jnp.exp(alpha - amax_e) / (den_e + 1e-16)
    xl_e = xl_g.reshape(-1, HEADS, CH)
    return (xl_e * w[:, :, None]).reshape(-1, HEADS * CH)


def _proj_kernel(xv_ref, xc_ref, wv_ref, wc_ref, b_ref, o_ref):
    o_ref[...] = (jnp.dot(xv_ref[...], wv_ref[...],
                          preferred_element_type=jnp.float32)
                  + jnp.dot(xc_ref[...], wc_ref[...],
                            preferred_element_type=jnp.float32)
                  + b_ref[...])


def _proj_pack(xv, xc, wv_list, wc_list, b_list):
    dv, dc = xv.shape[1], xc.shape[1]
    wv = jnp.zeros((dv, 256), jnp.float32)
    wc = jnp.zeros((dc, 256), jnp.float32)
    b = jnp.zeros((256,), jnp.float32)
    for i, (w, which) in enumerate(wv_list):
        if which == "v":
            wv = wv.at[:, i * 64:(i + 1) * 64].set(w)
        else:
            wc = wc.at[:, i * 64:(i + 1) * 64].set(w)
    for i, bb in enumerate(b_list):
        b = b.at[i * 64:(i + 1) * 64].set(bb)
    return pl.pallas_call(
        _proj_kernel,
        out_shape=jax.ShapeDtypeStruct((xv.shape[0], 256), jnp.float32),
    )(xv, xc, wv, wc, b.reshape(1, 256))


def _gn_kernel(ms_ref, w_ref, b_ref, x_ref, o_ref):
    x = x_ref[...]
    mean = jnp.mean(x, axis=0, keepdims=True)
    out = x - mean * ms_ref[...]
    var = jnp.mean(out * out, axis=0, keepdims=True)
    std = jnp.sqrt(var + 1e-5)
    y = w_ref[...] * (out / std) + b_ref[...]
    o_ref[...] = jnp.where(y >= 0, y, 0.01 * y)


def _graph_norm_lrelu(x, w, b, ms):
    d = x.shape[1]
    return pl.pallas_call(
        _gn_kernel,
        out_shape=jax.ShapeDtypeStruct(x.shape, jnp.float32),
    )(ms.reshape(1, d), w.reshape(1, d), b.reshape(1, d), x)


def kernel(objective_vector, variable_lower_bound, variable_upper_bound, constraint_lower_bound, edge_values, params, edge_index, vars_ptr, cons_ptr):
    lb = variable_lower_bound
    ub = variable_upper_bound
    lb = jnp.where(jnp.isposinf(lb), 100.0, lb)
    ub = jnp.where(jnp.isposinf(ub), 100.0, ub)
    lb = jnp.where(jnp.isneginf(lb), -100.0, lb)
    lb = jnp.where(jnp.isneginf(ub), -100.0, lb)
    x_vars = jnp.stack([objective_vector, lb, ub], axis=1)
    x_vars = _fourier(x_vars, LEVEL_VEC)
    x_cons = _fourier(constraint_lower_bound[:, None], LEVEL_CON)
    src_c = edge_index[0]
    dst_v = edge_index[1]
    zeros64 = jnp.zeros((NV, 64), jnp.float32)
    for layer in params["layers"]:
        lo, lc, lv = layer["obj"], layer["c2v"], layer["v2c"]
        p_src = _proj_pack(x_vars, x_cons,
                           [(lo["Wl"], "v"), (lc["Wl"], "c"), (lv["Wr"], "c")],
                           None,
                           [lo["bl"], lc["bl"], lv["br"]])
        p_dst = _proj_pack(x_vars, x_cons,
                           [(lo["Wr"], "v"), (lc["Wr"], "v"), (lv["Wl"], "v")],
                           None,
                           [lo["br"], lc["br"], lv["bl"]])
        g_src, g_dst = _gather_rows2(p_src, src_c, p_dst, dst_v, chunk=200)
        a_obj = _gat_alpha(g_src[:, 0:64], g_dst[:, 0:64], edge_values, lo)
        a_c2v = _gat_alpha(g_src[:, 64:128], g_dst[:, 64:128], edge_values, lc)
        a_v2c = _gat_alpha(g_dst[:, 128:192], g_src[:, 128:192], edge_values, lv)
        amax_o, den_o = _segment_softmax_stats(a_obj, dst_v, NV)
        amax_c, den_c = _segment_softmax_stats(a_c2v, dst_v, NV)
        amax_v, den_v = _segment_softmax_stats(a_v2c, src_c, NC)
        t_dst = jnp.concatenate([amax_o, den_o, amax_c, den_c,
                                 jnp.zeros((NV, 112), jnp.float32)], axis=1)
        t_src = jnp.concatenate([amax_v, den_v,
                                 jnp.zeros((NC, 120), jnp.float32)], axis=1)
        gmd_dst, gmd_src = _gather_rows2(t_dst, dst_v, t_src, src_c, chunk=400)
        y_obj = _gat_weight(g_src[:, 0:64], a_obj, gmd_dst[:, 0:4], gmd_dst[:, 4:8])
        y_c2v = _gat_weight(g_src[:, 64:128], a_c2v, gmd_dst[:, 8:12], gmd_dst[:, 12:16])
        y_v2c = _gat_weight(g_dst[:, 128:192], a_v2c, gmd_src[:, 0:4], gmd_src[:, 4:8])
        x_vars = jax.ops.segment_sum(y_obj + y_c2v, dst_v, num_segments=NV) + lo["bias"] + lc["bias"]
        x_cons = jax.ops.segment_sum(y_v2c, src_c, num_segments=NC) + lv["bias"]
        x_vars = _graph_norm_lrelu(x_vars, params["gn_weight"], params["gn_bias"], params["gn_mean_scale"])
        x_cons = _graph_norm_lrelu(x_cons, params["gn_weight"], params["gn_bias"], params["gn_mean_scale"])
    return (x_vars, x_cons)


# speedup vs baseline: 1.2243x; 1.0101x over previous
import functools
import math

import jax
import jax.numpy as jnp
from jax import lax
from jax.experimental import pallas as pl
from jax.experimental.pallas import tpu as pltpu
from jax.experimental.pallas import tpu_sc as plsc

NV = 10000
NC = 10000
E = 320000
HIDDEN = 64
HEADS = 4
CH = HIDDEN // HEADS
NUM_LAYERS = 2
LEVEL_VEC = math.ceil(HIDDEN / 6)
LEVEL_CON = math.ceil(HIDDEN / 2)


def _fourier(x, level):
    scales = 2.0 ** jnp.arange(-level / 2.0, level / 2.0, dtype=x.dtype)
    ms = jnp.concatenate([x / s for s in scales], axis=1)
    return jnp.concatenate([jnp.sin(ms), jnp.cos(ms)], axis=1)


try:
    _SC_INFO = plsc.get_sparse_core_info()
    _SC_CORES = _SC_INFO.num_cores
    _SC_SUBCORES = _SC_INFO.num_subcores
except Exception:
    _SC_CORES, _SC_SUBCORES = 2, 16
_NW = _SC_CORES * _SC_SUBCORES


def _gather2_kernel(per_w, chunk, d1, d2,
                    t1_hbm, i1_hbm, t2_hbm, i2_hbm, o1_hbm, o2_hbm,
                    i1_v, i2_v, r1_v, r2_v, sem1, sem2):
    wid = lax.axis_index("s") * _SC_CORES + lax.axis_index("c")
    base = wid * per_w

    def body(j, carry):
        off = base + j * chunk
        pltpu.sync_copy(i1_hbm.at[pl.ds(off, chunk)], i1_v)
        pltpu.sync_copy(i2_hbm.at[pl.ds(off, chunk)], i2_v)
        cp1 = pltpu.async_copy(t1_hbm.at[i1_v, :], r1_v, sem1)
        cp2 = pltpu.async_copy(t2_hbm.at[i2_v, :], r2_v, sem2)
        cp1.wait()
        cp2.wait()
        pltpu.sync_copy(r1_v, o1_hbm.at[pl.ds(off, chunk)])
        pltpu.sync_copy(r2_v, o2_hbm.at[pl.ds(off, chunk)])
        return carry

    lax.fori_loop(0, per_w // chunk, body, 0)


def _gather1_kernel(per_w, chunk, d1,
                    t1_hbm, i1_hbm, o1_hbm, i1_v, r1_v, sem1):
    wid = lax.axis_index("s") * _SC_CORES + lax.axis_index("c")
    base = wid * per_w

    def body(j, carry):
        off = base + j * chunk
        pltpu.sync_copy(i1_hbm.at[pl.ds(off, chunk)], i1_v)
        pltpu.async_copy(t1_hbm.at[i1_v, :], r1_v, sem1).wait()
        pltpu.sync_copy(r1_v, o1_hbm.at[pl.ds(off, chunk)])
        return carry

    lax.fori_loop(0, per_w // chunk, body, 0)


def _gather_rows1(t1, i1, chunk=400):
    e = i1.shape[0]
    per_w = e // _NW
    d1 = t1.shape[1]
    mesh = plsc.VectorSubcoreMesh(core_axis_name="c", subcore_axis_name="s", num_cores=_SC_CORES, num_subcores=_SC_SUBCORES)
    f = functools.partial(
        pl.kernel,
        out_type=jax.ShapeDtypeStruct((e, d1), jnp.float32),
        mesh=mesh,
        scratch_types=[
            pltpu.VMEM((chunk,), jnp.int32),
            pltpu.VMEM((chunk, d1), jnp.float32),
            pltpu.SemaphoreType.DMA,
        ],
    )(functools.partial(_gather1_kernel, per_w, chunk, d1))
    return f(t1, i1.astype(jnp.int32))


def _gather_rows2(t1, i1, t2, i2, chunk=1000):
    e = i1.shape[0]
    per_w = e // _NW
    d1 = t1.shape[1]
    d2 = t2.shape[1]
    mesh = plsc.VectorSubcoreMesh(core_axis_name="c", subcore_axis_name="s", num_cores=_SC_CORES, num_subcores=_SC_SUBCORES)
    f = functools.partial(
        pl.kernel,
        out_type=(jax.ShapeDtypeStruct((e, d1), jnp.float32),
                  jax.ShapeDtypeStruct((e, d2), jnp.float32)),
        mesh=mesh,
        scratch_types=[
            pltpu.VMEM((chunk,), jnp.int32),
            pltpu.VMEM((chunk,), jnp.int32),
            pltpu.VMEM((chunk, d1), jnp.float32),
            pltpu.VMEM((chunk, d2), jnp.float32),
            pltpu.SemaphoreType.DMA,
            pltpu.SemaphoreType.DMA,
        ],
    )(functools.partial(_gather2_kernel, per_w, chunk, d1, d2))
    return f(t1, i1.astype(jnp.int32), t2, i2.astype(jnp.int32))


_ACC_ROWS = 3328
_PASS_ROWS = 3200
_N_PASSES = 4


def _scatter_add_kernel(per_w, chunk, d,
                        ya_hbm, ia_hbm, yb_hbm, ib_hbm, out_hbm,
                        ia_v, ib_v, ia2_v, ib2_v, ra_v, rb_v, zbuf, acc, sem):
    c = lax.axis_index("c")
    s = lax.axis_index("s")
    wid = s * _SC_CORES + c
    base = wid * per_w
    rows_per_s = _ACC_ROWS // _SC_SUBCORES
    r0 = s * rows_per_s
    zv = jnp.zeros((16,), jnp.float32)

    def zbody(i, carry):
        for cc in range(d // 16):
            zbuf[i, pl.ds(cc * 16, 16)] = zv
        return carry

    lax.fori_loop(0, rows_per_s, zbody, 0)

    for p in range(_N_PASSES):
        lo = p * _PASS_ROWS
        pltpu.sync_copy(zbuf, acc.at[pl.ds(r0, rows_per_s), :])
        plsc.subcore_barrier()

        def body(j, carry):
            off = base + j * chunk
            pltpu.sync_copy(ia_hbm.at[pl.ds(off, chunk)], ia_v)
            pltpu.sync_copy(ib_hbm.at[pl.ds(off, chunk)], ib_v)
            pltpu.sync_copy(ya_hbm.at[pl.ds(off, chunk), :], ra_v)
            pltpu.sync_copy(yb_hbm.at[pl.ds(off, chunk), :], rb_v)
            for k in range(chunk // 16):
                va = ia_v[pl.ds(k * 16, 16)] - lo
                vb = ib_v[pl.ds(k * 16, 16)] - lo
                oka = (va >= 0) & (va < _PASS_ROWS)
                okb = (vb >= 0) & (vb < _PASS_ROWS)
                ia2_v[pl.ds(k * 16, 16)] = jnp.where(oka, va, _ACC_ROWS - 8)
                ib2_v[pl.ds(k * 16, 16)] = jnp.where(okb, vb, _ACC_ROWS - 8)
            pltpu.sync_copy(ra_v, acc.at[ia2_v, :], add=True)
            pltpu.sync_copy(rb_v, acc.at[ib2_v, :], add=True)
            return carry

        lax.fori_loop(0, per_w // chunk, body, 0)
        plsc.subcore_barrier()
        pltpu.sync_copy(acc.at[pl.ds(r0, rows_per_s), :],
                        out_hbm.at[c, p, pl.ds(r0, rows_per_s), :])
        plsc.subcore_barrier()


def _scatter_add_dual(y_a, idx_a, y_b, idx_b, chunk=400):
    e, d = y_a.shape
    per_w = e // _NW
    mesh = plsc.VectorSubcoreMesh(core_axis_name="c", subcore_axis_name="s", num_cores=_SC_CORES, num_subcores=_SC_SUBCORES)
    f = functools.partial(
        pl.kernel,
        out_type=pltpu.HBM((_SC_CORES, _N_PASSES, _ACC_ROWS, d), jnp.float32),
        mesh=mesh,
        scratch_types=[
            pltpu.VMEM((chunk,), jnp.int32),
            pltpu.VMEM((chunk,), jnp.int32),
            pltpu.VMEM((chunk,), jnp.int32),
            pltpu.VMEM((chunk,), jnp.int32),
            pltpu.VMEM((chunk, d), jnp.float32),
            pltpu.VMEM((chunk, d), jnp.float32),
            pltpu.VMEM((_ACC_ROWS // _SC_SUBCORES, d), jnp.float32),
            pltpu.VMEM_SHARED((_ACC_ROWS, d), jnp.float32),
            pltpu.SemaphoreType.DMA,
        ],
        compiler_params=pltpu.CompilerParams(needs_layout_passes=False),
    )(functools.partial(_scatter_add_kernel, per_w, chunk, d))
    part = f(y_a, idx_a.astype(jnp.int32), y_b, idx_b.astype(jnp.int32))
    return jnp.concatenate([part[:, p, :_PASS_ROWS, :] for p in range(_N_PASSES)], axis=1)


def _merge2_kernel(n_rows, c0, p_ref, b_ref, o_ref):
    o_ref[...] = (p_ref[0, :n_rows, c0:c0 + HIDDEN]
                  + p_ref[1, :n_rows, c0:c0 + HIDDEN] + b_ref[...])


def _merge2_bias(part, bias, n_rows, c0):
    return pl.pallas_call(
        functools.partial(_merge2_kernel, n_rows, c0),
        out_shape=jax.ShapeDtypeStruct((n_rows, HIDDEN), jnp.float32),
    )(part, bias.reshape(1, HIDDEN))


_NEG = -3.0e38


def _mdmerge_kernel(m_ref, d_ref, amax_ref, dg_ref):
    m = m_ref[...].reshape(_NW, -1)
    d = d_ref[...].reshape(_NW, -1)
    m_g = jnp.max(m, axis=0)
    scale = jnp.where(d > 0, jnp.exp(m - m_g[None]), 0.0)
    dg_ref[...] = jnp.sum(d * scale, axis=0)
    amax_ref[...] = jnp.where(m_g > -1.0e37, m_g, 0.0)


def _soft_kernel(per_w, chunk, n4,
                 al_hbm, dst_hbm, m_out, d_out,
                 dst_b, al_b, m_priv, d_priv):
    c = lax.axis_index("c")
    s = lax.axis_index("s")
    wid = s * _SC_CORES + c
    base = wid * per_w
    iota = lax.iota(jnp.int32, 16)
    mask4 = iota < 4
    sel4 = jnp.minimum(iota, 3)
    negv = jnp.full((16,), _NEG, jnp.float32)
    zv = jnp.zeros((16,), jnp.float32)

    def initb(i, carry):
        for k in range(16):
            m_priv[i, pl.ds(k * 16, 16)] = negv
            d_priv[i, pl.ds(k * 16, 16)] = zv
        return carry

    lax.fori_loop(0, n4 // 256, initb, 0)

    def load_chunk(g):
        off = base + g * chunk
        pltpu.sync_copy(dst_hbm.at[pl.ds(off, chunk)], dst_b)
        pltpu.sync_copy(al_hbm.at[pl.ds(off * 4, chunk * 4)], al_b)

    rot4 = jnp.bitwise_and(iota + 4, 15)
    rot8 = jnp.bitwise_and(iota + 8, 15)
    g4 = lax.shift_right_logical(iota, 2)
    h4 = jnp.bitwise_and(iota, 3)

    def edge_quad(gg, do_pass2):
        dstv = dst_b[pl.ds(gg * 16, 16)]

        for q in range(4):
            dq = dstv.at[q * 4 + g4].get(mode="promise_in_bounds")
            aq = al_b[pl.ds(gg * 64 + q * 16, 16)]
            idx = dq * 4 + h4
            idr = lax.shift_right_logical(idx, 8)
            idc = jnp.bitwise_and(idx, 255)
            c1 = dq == dq.at[rot4].get(mode="promise_in_bounds")
            c2 = dq == dq.at[rot8].get(mode="promise_in_bounds")
            conflict = jnp.any(c1 | c2)

            def rmw(mask):
                if do_pass2:
                    mcur = plsc.load_gather(m_priv, [idr, idc])
                    e = jnp.exp(aq - mcur)
                    dcur = plsc.load_gather(d_priv, [idr, idc])
                    plsc.store_scatter(d_priv, [idr, idc], dcur + e,
                                       mask=mask)
                else:
                    mcur = plsc.load_gather(m_priv, [idr, idc])
                    plsc.store_scatter(m_priv, [idr, idc],
                                       jnp.maximum(mcur, aq), mask=mask)

            def fast(carry):
                rmw(None)
                return carry

            def slow(carry):
                for k in range(4):
                    rmw(g4 == k)
                return carry

            lax.cond(conflict, slow, fast, 0)

    def pass1(g, carry):
        load_chunk(g)

        def inner(gg, cc):
            edge_quad(gg, False)
            return cc

        lax.fori_loop(0, chunk // 16, inner, 0)
        return carry

    def pass2(g, carry):
        load_chunk(g)

        def inner(gg, cc):
            edge_quad(gg, True)
            return cc

        lax.fori_loop(0, chunk // 16, inner, 0)
        return carry

    lax.fori_loop(0, per_w // chunk, pass1, 0)
    lax.fori_loop(0, per_w // chunk, pass2, 0)
    pltpu.sync_copy(m_priv, m_out.at[c, s])
    pltpu.sync_copy(d_priv, d_out.at[c, s])


def _segment_softmax_stats(alpha, dst, n_rows, chunk=2000):
    e = alpha.shape[0]
    per_w = e // _NW
    n_pad = ((n_rows + 8 * _SC_SUBCORES - 1) // (8 * _SC_SUBCORES)) * 8 * _SC_SUBCORES
    n4 = n_pad * 4
    mesh = plsc.VectorSubcoreMesh(core_axis_name="c", subcore_axis_name="s",
                                  num_cores=_SC_CORES, num_subcores=_SC_SUBCORES)
    f = functools.partial(
        pl.kernel,
        out_type=(pltpu.HBM((_SC_CORES, _SC_SUBCORES, n4 // 256, 256), jnp.float32),
                  pltpu.HBM((_SC_CORES, _SC_SUBCORES, n4 // 256, 256), jnp.float32)),
        mesh=mesh,
        scratch_types=[
            pltpu.VMEM((chunk,), jnp.int32),
            pltpu.VMEM((chunk * 4,), jnp.float32),
            pltpu.VMEM((n4 // 256, 256), jnp.float32),
            pltpu.VMEM((n4 // 256, 256), jnp.float32),
        ],
        compiler_params=pltpu.CompilerParams(needs_layout_passes=False),
    )(functools.partial(_soft_kernel, per_w, chunk, n4))
    alpha_flat = alpha.reshape(-1)
    m_all, d_all = f(alpha_flat, dst.astype(jnp.int32))
    amax, d_g = pl.pallas_call(
        _mdmerge_kernel,
        out_shape=(jax.ShapeDtypeStruct((n4,), jnp.float32),
                   jax.ShapeDtypeStruct((n4,), jnp.float32)),
    )(m_all, d_all)
    amax = amax.reshape(n_pad, 4)
    d_g = d_g.reshape(n_pad, 4)
    return amax[:n_rows], d_g[:n_rows]


def _gat_alpha(xl_g, xr_g, edge_attr, p):
    xl_e = xl_g.reshape(-1, HEADS, CH)
    xr_e = xr_g.reshape(-1, HEADS, CH)
    ee = (edge_attr[:, None] @ p["We"]).reshape(-1, HEADS, CH)
    m = jax.nn.leaky_relu(xl_e + xr_e + ee, 0.2)
    return jnp.sum(m * p["att"][None], axis=-1)


def _gat_weight(xl_g, alpha, amax_e, den_e):
    w = jnp.exp(alpha - amax_e) / (den_e + 1e-16)
    xl_e = xl_g.reshape(-1, HEADS, CH)
    return (xl_e * w[:, :, None]).reshape(-1, HEADS * CH)


def _proj_kernel(xv_ref, xc_ref, wv_ref, wc_ref, b_ref, o_ref):
    o_ref[...] = (jnp.dot(xv_ref[...], wv_ref[...],
                          preferred_element_type=jnp.float32)
                  + jnp.dot(xc_ref[...], wc_ref[...],
                            preferred_element_type=jnp.float32)
                  + b_ref[...])


def _proj_pack(xv, xc, wv_list, wc_list, b_list):
    dv, dc = xv.shape[1], xc.shape[1]
    wv = jnp.zeros((dv, 256), jnp.float32)
    wc = jnp.zeros((dc, 256), jnp.float32)
    b = jnp.zeros((256,), jnp.float32)
    for i, (w, which) in enumerate(wv_list):
        if which == "v":
            wv = wv.at[:, i * 64:(i + 1) * 64].set(w)
        else:
            wc = wc.at[:, i * 64:(i + 1) * 64].set(w)
    for i, bb in enumerate(b_list):
        b = b.at[i * 64:(i + 1) * 64].set(bb)
    return pl.pallas_call(
        _proj_kernel,
        out_shape=jax.ShapeDtypeStruct((xv.shape[0], 256), jnp.float32),
    )(xv, xc, wv, wc, b.reshape(1, 256))


def _gn_kernel(ms_ref, w_ref, b_ref, x_ref, o_ref):
    x = x_ref[...]
    mean = jnp.mean(x, axis=0, keepdims=True)
    out = x - mean * ms_ref[...]
    var = jnp.mean(out * out, axis=0, keepdims=True)
    std = jnp.sqrt(var + 1e-5)
    y = w_ref[...] * (out / std) + b_ref[...]
    o_ref[...] = jnp.where(y >= 0, y, 0.01 * y)


def _graph_norm_lrelu(x, w, b, ms):
    d = x.shape[1]
    return pl.pallas_call(
        _gn_kernel,
        out_shape=jax.ShapeDtypeStruct(x.shape, jnp.float32),
    )(ms.reshape(1, d), w.reshape(1, d), b.reshape(1, d), x)


def kernel(objective_vector, variable_lower_bound, variable_upper_bound, constraint_lower_bound, edge_values, params, edge_index, vars_ptr, cons_ptr):
    lb = variable_lower_bound
    ub = variable_upper_bound
    lb = jnp.where(jnp.isposinf(lb), 100.0, lb)
    ub = jnp.where(jnp.isposinf(ub), 100.0, ub)
    lb = jnp.where(jnp.isneginf(lb), -100.0, lb)
    lb = jnp.where(jnp.isneginf(ub), -100.0, lb)
    x_vars = jnp.stack([objective_vector, lb, ub], axis=1)
    x_vars = _fourier(x_vars, LEVEL_VEC)
    x_cons = _fourier(constraint_lower_bound[:, None], LEVEL_CON)
    src_c = edge_index[0]
    dst_v = edge_index[1]
    zeros64 = jnp.zeros((NV, 64), jnp.float32)
    for layer in params["layers"]:
        lo, lc, lv = layer["obj"], layer["c2v"], layer["v2c"]
        p_src = _proj_pack(x_vars, x_cons,
                           [(lo["Wl"], "v"), (lc["Wl"], "c"), (lv["Wr"], "c")],
                           None,
                           [lo["bl"], lc["bl"], lv["br"]])
        p_dst = _proj_pack(x_vars, x_cons,
                           [(lo["Wr"], "v"), (lc["Wr"], "v"), (lv["Wl"], "v")],
                           None,
                           [lo["br"], lc["br"], lv["bl"]])
        g_src = _gather_rows1(p_src, src_c)
        g_dst = _gather_rows1(p_dst, dst_v)
        a_obj = _gat_alpha(g_src[:, 0:64], g_dst[:, 0:64], edge_values, lo)
        a_c2v = _gat_alpha(g_src[:, 64:128], g_dst[:, 64:128], edge_values, lc)
        a_v2c = _gat_alpha(g_dst[:, 128:192], g_src[:, 128:192], edge_values, lv)
        amax_o, den_o = _segment_softmax_stats(a_obj, dst_v, NV)
        amax_c, den_c = _segment_softmax_stats(a_c2v, dst_v, NV)
        amax_v, den_v = _segment_softmax_stats(a_v2c, src_c, NC)
        t_dst = jnp.concatenate([amax_o, den_o, amax_c, den_c,
                                 jnp.zeros((NV, 112), jnp.float32)], axis=1)
        t_src = jnp.concatenate([amax_v, den_v,
                                 jnp.zeros((NC, 120), jnp.float32)], axis=1)
        gmd_dst, gmd_src = _gather_rows2(t_dst, dst_v, t_src, src_c, chunk=400)
        y_obj = _gat_weight(g_src[:, 0:64], a_obj, gmd_dst[:, 0:4], gmd_dst[:, 4:8])
        y_c2v = _gat_weight(g_src[:, 64:128], a_c2v, gmd_dst[:, 8:12], gmd_dst[:, 12:16])
        y_v2c = _gat_weight(g_dst[:, 128:192], a_v2c, gmd_src[:, 0:4], gmd_src[:, 4:8])
        x_vars = jax.ops.segment_sum(y_obj + y_c2v, dst_v, num_segments=NV) + lo["bias"] + lc["bias"]
        x_cons = jax.ops.segment_sum(y_v2c, src_c, num_segments=NC) + lv["bias"]
        x_vars = _graph_norm_lrelu(x_vars, params["gn_weight"], params["gn_bias"], params["gn_mean_scale"])
        x_cons = _graph_norm_lrelu(x_cons, params["gn_weight"], params["gn_bias"], params["gn_mean_scale"])
    return (x_vars, x_cons)


# speedup vs baseline: 1.2247x; 1.0003x over previous
import functools
import math

import jax
import jax.numpy as jnp
from jax import lax
from jax.experimental import pallas as pl
from jax.experimental.pallas import tpu as pltpu
from jax.experimental.pallas import tpu_sc as plsc

NV = 10000
NC = 10000
E = 320000
HIDDEN = 64
HEADS = 4
CH = HIDDEN // HEADS
NUM_LAYERS = 2
LEVEL_VEC = math.ceil(HIDDEN / 6)
LEVEL_CON = math.ceil(HIDDEN / 2)


def _fourier(x, level):
    scales = 2.0 ** jnp.arange(-level / 2.0, level / 2.0, dtype=x.dtype)
    ms = jnp.concatenate([x / s for s in scales], axis=1)
    return jnp.concatenate([jnp.sin(ms), jnp.cos(ms)], axis=1)


try:
    _SC_INFO = plsc.get_sparse_core_info()
    _SC_CORES = _SC_INFO.num_cores
    _SC_SUBCORES = _SC_INFO.num_subcores
except Exception:
    _SC_CORES, _SC_SUBCORES = 2, 16
_NW = _SC_CORES * _SC_SUBCORES


def _gather2_kernel(per_w, chunk, d1, d2,
                    t1_hbm, i1_hbm, t2_hbm, i2_hbm, o1_hbm, o2_hbm,
                    i1_v, i2_v, r1_v, r2_v, sem1, sem2):
    wid = lax.axis_index("s") * _SC_CORES + lax.axis_index("c")
    base = wid * per_w

    def body(j, carry):
        off = base + j * chunk
        pltpu.sync_copy(i1_hbm.at[pl.ds(off, chunk)], i1_v)
        pltpu.sync_copy(i2_hbm.at[pl.ds(off, chunk)], i2_v)
        cp1 = pltpu.async_copy(t1_hbm.at[i1_v, :], r1_v, sem1)
        cp2 = pltpu.async_copy(t2_hbm.at[i2_v, :], r2_v, sem2)
        cp1.wait()
        cp2.wait()
        pltpu.sync_copy(r1_v, o1_hbm.at[pl.ds(off, chunk)])
        pltpu.sync_copy(r2_v, o2_hbm.at[pl.ds(off, chunk)])
        return carry

    lax.fori_loop(0, per_w // chunk, body, 0)


def _gather1_kernel(per_w, chunk, d1,
                    t1_hbm, i1_hbm, o1_hbm, i1_v, r1_v, sem1):
    wid = lax.axis_index("s") * _SC_CORES + lax.axis_index("c")
    base = wid * per_w

    def body(j, carry):
        off = base + j * chunk
        pltpu.sync_copy(i1_hbm.at[pl.ds(off, chunk)], i1_v)
        pltpu.async_copy(t1_hbm.at[i1_v, :], r1_v, sem1).wait()
        pltpu.sync_copy(r1_v, o1_hbm.at[pl.ds(off, chunk)])
        return carry

    lax.fori_loop(0, per_w // chunk, body, 0)


def _gather_rows1(t1, i1, chunk=400):
    e = i1.shape[0]
    per_w = e // _NW
    d1 = t1.shape[1]
    mesh = plsc.VectorSubcoreMesh(core_axis_name="c", subcore_axis_name="s", num_cores=_SC_CORES, num_subcores=_SC_SUBCORES)
    f = functools.partial(
        pl.kernel,
        out_type=jax.ShapeDtypeStruct((e, d1), jnp.float32),
        mesh=mesh,
        scratch_types=[
            pltpu.VMEM((chunk,), jnp.int32),
            pltpu.VMEM((chunk, d1), jnp.float32),
            pltpu.SemaphoreType.DMA,
        ],
    )(functools.partial(_gather1_kernel, per_w, chunk, d1))
    return f(t1, i1.astype(jnp.int32))


def _gather_rows2(t1, i1, t2, i2, chunk=1000):
    e = i1.shape[0]
    per_w = e // _NW
    d1 = t1.shape[1]
    d2 = t2.shape[1]
    mesh = plsc.VectorSubcoreMesh(core_axis_name="c", subcore_axis_name="s", num_cores=_SC_CORES, num_subcores=_SC_SUBCORES)
    f = functools.partial(
        pl.kernel,
        out_type=(jax.ShapeDtypeStruct((e, d1), jnp.float32),
                  jax.ShapeDtypeStruct((e, d2), jnp.float32)),
        mesh=mesh,
        scratch_types=[
            pltpu.VMEM((chunk,), jnp.int32),
            pltpu.VMEM((chunk,), jnp.int32),
            pltpu.VMEM((chunk, d1), jnp.float32),
            pltpu.VMEM((chunk, d2), jnp.float32),
            pltpu.SemaphoreType.DMA,
            pltpu.SemaphoreType.DMA,
        ],
    )(functools.partial(_gather2_kernel, per_w, chunk, d1, d2))
    return f(t1, i1.astype(jnp.int32), t2, i2.astype(jnp.int32))


_NEG = -3.0e38


def _mdmerge_kernel(m_ref, d_ref, amax_ref, dg_ref):
    m = m_ref[...].reshape(_NW, -1)
    d = d_ref[...].reshape(_NW, -1)
    m_g = jnp.max(m, axis=0)
    scale = jnp.where(d > 0, jnp.exp(m - m_g[None]), 0.0)
    dg_ref[...] = jnp.sum(d * scale, axis=0)
    amax_ref[...] = jnp.where(m_g > -1.0e37, m_g, 0.0)


def _soft_kernel(per_w, chunk, n4,
                 al_hbm, dst_hbm, m_out, d_out,
                 dst_b, al_b, m_priv, d_priv):
    c = lax.axis_index("c")
    s = lax.axis_index("s")
    wid = s * _SC_CORES + c
    base = wid * per_w
    iota = lax.iota(jnp.int32, 16)
    mask4 = iota < 4
    sel4 = jnp.minimum(iota, 3)
    negv = jnp.full((16,), _NEG, jnp.float32)
    zv = jnp.zeros((16,), jnp.float32)

    def initb(i, carry):
        for k in range(16):
            m_priv[i, pl.ds(k * 16, 16)] = negv
            d_priv[i, pl.ds(k * 16, 16)] = zv
        return carry

    lax.fori_loop(0, n4 // 256, initb, 0)

    def load_chunk(g):
        off = base + g * chunk
        pltpu.sync_copy(dst_hbm.at[pl.ds(off, chunk)], dst_b)
        pltpu.sync_copy(al_hbm.at[pl.ds(off * 4, chunk * 4)], al_b)

    rot4 = jnp.bitwise_and(iota + 4, 15)
    rot8 = jnp.bitwise_and(iota + 8, 15)
    g4 = lax.shift_right_logical(iota, 2)
    h4 = jnp.bitwise_and(iota, 3)

    def edge_quad(gg, do_pass2):
        dstv = dst_b[pl.ds(gg * 16, 16)]

        for q in range(4):
            dq = dstv.at[q * 4 + g4].get(mode="promise_in_bounds")
            aq = al_b[pl.ds(gg * 64 + q * 16, 16)]
            idx = dq * 4 + h4
            idr = lax.shift_right_logical(idx, 8)
            idc = jnp.bitwise_and(idx, 255)
            c1 = dq == dq.at[rot4].get(mode="promise_in_bounds")
            c2 = dq == dq.at[rot8].get(mode="promise_in_bounds")
            conflict = jnp.any(c1 | c2)

            def rmw(mask):
                if do_pass2:
                    mcur = plsc.load_gather(m_priv, [idr, idc])
                    e = jnp.exp(aq - mcur)
                    dcur = plsc.load_gather(d_priv, [idr, idc])
                    plsc.store_scatter(d_priv, [idr, idc], dcur + e,
                                       mask=mask)
                else:
                    mcur = plsc.load_gather(m_priv, [idr, idc])
                    plsc.store_scatter(m_priv, [idr, idc],
                                       jnp.maximum(mcur, aq), mask=mask)

            def fast(carry):
                rmw(None)
                return carry

            def slow(carry):
                for k in range(4):
                    rmw(g4 == k)
                return carry

            lax.cond(conflict, slow, fast, 0)

    def pass1(g, carry):
        load_chunk(g)

        def inner(gg, cc):
            edge_quad(gg, False)
            return cc

        lax.fori_loop(0, chunk // 16, inner, 0)
        return carry

    def pass2(g, carry):
        load_chunk(g)

        def inner(gg, cc):
            edge_quad(gg, True)
            return cc

        lax.fori_loop(0, chunk // 16, inner, 0)
        return carry

    lax.fori_loop(0, per_w // chunk, pass1, 0)
    lax.fori_loop(0, per_w // chunk, pass2, 0)
    pltpu.sync_copy(m_priv, m_out.at[c, s])
    pltpu.sync_copy(d_priv, d_out.at[c, s])


def _segment_softmax_stats(alpha, dst, n_rows, chunk=2000):
    e = alpha.shape[0]
    per_w = e // _NW
    n_pad = ((n_rows + 8 * _SC_SUBCORES - 1) // (8 * _SC_SUBCORES)) * 8 * _SC_SUBCORES
    n4 = n_pad * 4
    mesh = plsc.VectorSubcoreMesh(core_axis_name="c", subcore_axis_name="s",
                                  num_cores=_SC_CORES, num_subcores=_SC_SUBCORES)
    f = functools.partial(
        pl.kernel,
        out_type=(pltpu.HBM((_SC_CORES, _SC_SUBCORES, n4 // 256, 256), jnp.float32),
                  pltpu.HBM((_SC_CORES, _SC_SUBCORES, n4 // 256, 256), jnp.float32)),
        mesh=mesh,
        scratch_types=[
            pltpu.VMEM((chunk,), jnp.int32),
            pltpu.VMEM((chunk * 4,), jnp.float32),
            pltpu.VMEM((n4 // 256, 256), jnp.float32),
            pltpu.VMEM((n4 // 256, 256), jnp.float32),
        ],
        compiler_params=pltpu.CompilerParams(needs_layout_passes=False),
    )(functools.partial(_soft_kernel, per_w, chunk, n4))
    alpha_flat = alpha.reshape(-1)
    m_all, d_all = f(alpha_flat, dst.astype(jnp.int32))
    amax, d_g = pl.pallas_call(
        _mdmerge_kernel,
        out_shape=(jax.ShapeDtypeStruct((n4,), jnp.float32),
                   jax.ShapeDtypeStruct((n4,), jnp.float32)),
    )(m_all, d_all)
    amax = amax.reshape(n_pad, 4)
    d_g = d_g.reshape(n_pad, 4)
    return amax[:n_rows], d_g[:n_rows]


def _gat_alpha(xl_g, xr_g, edge_attr, p):
    xl_e = xl_g.reshape(-1, HEADS, CH)
    xr_e = xr_g.reshape(-1, HEADS, CH)
    ee = (edge_attr[:, None] @ p["We"]).reshape(-1, HEADS, CH)
    m = jax.nn.leaky_relu(xl_e + xr_e + ee, 0.2)
    return jnp.sum(m * p["att"][None], axis=-1)


def _gat_weight(xl_g, alpha, amax_e, den_e):
    w = jnp.exp(alpha - amax_e) / (den_e + 1e-16)
    xl_e = xl_g.reshape(-1, HEADS, CH)
    return (xl_e * w[:, :, None]).reshape(-1, HEADS * CH)


def _proj_kernel(xv_ref, xc_ref, wv_ref, wc_ref, b_ref, o_ref):
    o_ref[...] = (jnp.dot(xv_ref[...], wv_ref[...],
                          preferred_element_type=jnp.float32)
                  + jnp.dot(xc_ref[...], wc_ref[...],
                            preferred_element_type=jnp.float32)
                  + b_ref[...])


def _proj_pack(xv, xc, wv_list, wc_list, b_list):
    dv, dc = xv.shape[1], xc.shape[1]
    wv = jnp.zeros((dv, 256), jnp.float32)
    wc = jnp.zeros((dc, 256), jnp.float32)
    b = jnp.zeros((256,), jnp.float32)
    for i, (w, which) in enumerate(wv_list):
        if which == "v":
            wv = wv.at[:, i * 64:(i + 1) * 64].set(w)
        else:
            wc = wc.at[:, i * 64:(i + 1) * 64].set(w)
    for i, bb in enumerate(b_list):
        b = b.at[i * 64:(i + 1) * 64].set(bb)
    return pl.pallas_call(
        _proj_kernel,
        out_shape=jax.ShapeDtypeStruct((xv.shape[0], 256), jnp.float32),
    )(xv, xc, wv, wc, b.reshape(1, 256))


def _gn_kernel(ms_ref, w_ref, b_ref, x_ref, o_ref):
    x = x_ref[...]
    mean = jnp.mean(x, axis=0, keepdims=True)
    out = x - mean * ms_ref[...]
    var = jnp.mean(out * out, axis=0, keepdims=True)
    std = jnp.sqrt(var + 1e-5)
    y = w_ref[...] * (out / std) + b_ref[...]
    o_ref[...] = jnp.where(y >= 0, y, 0.01 * y)


def _graph_norm_lrelu(x, w, b, ms):
    d = x.shape[1]
    return pl.pallas_call(
        _gn_kernel,
        out_shape=jax.ShapeDtypeStruct(x.shape, jnp.float32),
    )(ms.reshape(1, d), w.reshape(1, d), b.reshape(1, d), x)


def kernel(objective_vector, variable_lower_bound, variable_upper_bound, constraint_lower_bound, edge_values, params, edge_index, vars_ptr, cons_ptr):
    lb = variable_lower_bound
    ub = variable_upper_bound
    lb = jnp.where(jnp.isposinf(lb), 100.0, lb)
    ub = jnp.where(jnp.isposinf(ub), 100.0, ub)
    lb = jnp.where(jnp.isneginf(lb), -100.0, lb)
    lb = jnp.where(jnp.isneginf(ub), -100.0, lb)
    x_vars = jnp.stack([objective_vector, lb, ub], axis=1)
    x_vars = _fourier(x_vars, LEVEL_VEC)
    x_cons = _fourier(constraint_lower_bound[:, None], LEVEL_CON)
    src_c = edge_index[0]
    dst_v = edge_index[1]
    zeros64 = jnp.zeros((NV, 64), jnp.float32)
    for layer in params["layers"]:
        lo, lc, lv = layer["obj"], layer["c2v"], layer["v2c"]
        p_src = _proj_pack(x_vars, x_cons,
                           [(lo["Wl"], "v"), (lc["Wl"], "c"), (lv["Wr"], "c")],
                           None,
                           [lo["bl"], lc["bl"], lv["br"]])
        p_dst = _proj_pack(x_vars, x_cons,
                           [(lo["Wr"], "v"), (lc["Wr"], "v"), (lv["Wl"], "v")],
                           None,
                           [lo["br"], lc["br"], lv["bl"]])
        g_src = _gather_rows1(p_src, src_c)
        g_dst = _gather_rows1(p_dst, dst_v)
        a_obj = _gat_alpha(g_src[:, 0:64], g_dst[:, 0:64], edge_values, lo)
        a_c2v = _gat_alpha(g_src[:, 64:128], g_dst[:, 64:128], edge_values, lc)
        a_v2c = _gat_alpha(g_dst[:, 128:192], g_src[:, 128:192], edge_values, lv)
        amax_o, den_o = _segment_softmax_stats(a_obj, dst_v, NV)
        amax_c, den_c = _segment_softmax_stats(a_c2v, dst_v, NV)
        amax_v, den_v = _segment_softmax_stats(a_v2c, src_c, NC)
        t_dst = jnp.concatenate([amax_o, den_o, amax_c, den_c,
                                 jnp.zeros((NV, 112), jnp.float32)], axis=1)
        t_src = jnp.concatenate([amax_v, den_v,
                                 jnp.zeros((NC, 120), jnp.float32)], axis=1)
        gmd_dst, gmd_src = _gather_rows2(t_dst, dst_v, t_src, src_c, chunk=400)
        y_obj = _gat_weight(g_src[:, 0:64], a_obj, gmd_dst[:, 0:4], gmd_dst[:, 4:8])
        y_c2v = _gat_weight(g_src[:, 64:128], a_c2v, gmd_dst[:, 8:12], gmd_dst[:, 12:16])
        y_v2c = _gat_weight(g_dst[:, 128:192], a_v2c, gmd_src[:, 0:4], gmd_src[:, 4:8])
        x_vars = jax.ops.segment_sum(y_obj + y_c2v, dst_v, num_segments=NV) + lo["bias"] + lc["bias"]
        x_cons = jax.ops.segment_sum(y_v2c, src_c, num_segments=NC) + lv["bias"]
        x_vars = _graph_norm_lrelu(x_vars, params["gn_weight"], params["gn_bias"], params["gn_mean_scale"])
        x_cons = _graph_norm_lrelu(x_cons, params["gn_weight"], params["gn_bias"], params["gn_mean_scale"])
    return (x_vars, x_cons)
